# baseline (speedup 1.0000x reference)
import functools

import jax
import jax.numpy as jnp
from jax import lax
from jax.experimental import pallas as pl
from jax.experimental.pallas import tpu as pltpu
from jax.experimental.pallas import tpu_sc as plsc

_MARGIN = 1.0
_NUM_CLASSES = 10
_B = 256
_BIG = 3.0e38
_NEG = -3.0e38

_NC = 2
_NS = 16
_L = 16
_NW = _NC * _NS
_NPW = _B // _NW
_DP_LANE = _NUM_CLASSES


def _dense_kernel(x_ref, lab_row_ref, lab_col_ref,
                  d_ref, nd_ref, oc_ref, ord_ref):
    x = x_ref[:, :]
    lab_row = lab_row_ref[:, :]
    lab_col = lab_col_ref[:, :]
    B = x.shape[0]

    g = jax.lax.dot_general(
        x, x, (((1,), (1,)), ((), ())), preferred_element_type=jnp.float32
    )
    eye = (
        jax.lax.broadcasted_iota(jnp.int32, (B, B), 0)
        == jax.lax.broadcasted_iota(jnp.int32, (B, B), 1)
    )
    diag = jnp.where(eye, g, 0.0)
    n_col = jnp.sum(diag, axis=1, keepdims=True)
    n_row = jnp.sum(diag, axis=0, keepdims=True)
    d = n_col + n_row - 2.0 * g
    d_ref[:, :] = d

    same = lab_col == lab_row
    lt = lab_col < lab_row
    gt = lab_col > lab_row

    dp = jnp.max(jnp.where(same, d, _NEG), axis=1, keepdims=True)

    s_row = jnp.sum(d, axis=1, keepdims=True)
    cols = [None] * _L
    for c in range(_NUM_CLASSES):
        cs_c = jnp.sum(jnp.where(lab_row == c, d, 0.0), axis=1, keepdims=True)
        cols[c] = s_row - cs_c
    cols[_DP_LANE] = dp
    for c in range(_DP_LANE + 1, _L):
        cols[c] = jnp.full((B, 1), _BIG, jnp.float32)
    nd_ref[:, :] = jnp.concatenate(cols, axis=1)

    off_col = jnp.sum(gt.astype(jnp.int32), axis=1, keepdims=True)
    cnt_col = jnp.sum(same.astype(jnp.int32), axis=1, keepdims=True)
    zeros = jnp.zeros((B, 1), jnp.int32)
    oc_ref[:, :] = jnp.concatenate([off_col, cnt_col] + [zeros] * (_L - 2),
                                   axis=1)

    idx_row = jax.lax.broadcasted_iota(jnp.int32, (B, B), 1)
    idx_col = jax.lax.broadcasted_iota(jnp.int32, (B, B), 0)
    before = jnp.logical_or(lt, jnp.logical_and(same, idx_col < idx_row))
    pos_row = jnp.sum(before.astype(jnp.int32), axis=0, keepdims=True)

    sel = pos_row == idx_col
    ord_ref[:, :] = jnp.sum(jnp.where(sel, idx_row, 0), axis=1, keepdims=True)


def _mine_body(d_hbm, nd_hbm, oc_hbm, ord_hbm, out_hbm,
               ndrows, ocrows, ordv, idxb, dnb, outv, sem):
    c = lax.axis_index("c")
    s = lax.axis_index("s")
    wid = s * _NC + c
    base = wid * _NPW

    cp1 = pltpu.make_async_copy(nd_hbm.at[pl.ds(base, _NPW)], ndrows, sem)
    cp2 = pltpu.make_async_copy(oc_hbm.at[pl.ds(base, _NPW)], ocrows, sem)
    cp3 = pltpu.make_async_copy(ord_hbm, ordv, sem)
    cp1.start()
    cp2.start()
    cp3.start()
    cp1.wait()
    cp2.wait()
    cp3.wait()

    lanes = jax.lax.broadcasted_iota(jnp.int32, (_L,), 0)
    dp_acc = jnp.zeros((_L,), jnp.float32)
    idx_acc = jnp.zeros((_L,), jnp.int32)

    for a in range(_NPW):
        av = jnp.full((_L,), a, dtype=jnp.int32)
        ndv = ndrows[a, pl.ds(0, _L)]
        neg_v = jnp.where(lanes < _NUM_CLASSES, ndv, _BIG)
        minv = jnp.min(neg_v)
        m_v = plsc.all_reduce_ffs(neg_v == minv)
        off_a = plsc.load_gather(ocrows, [av, jnp.zeros((_L,), jnp.int32)])
        cnt_a = plsc.load_gather(ocrows, [av, jnp.ones((_L,), jnp.int32)])
        pv = m_v + jnp.where(m_v >= off_a, cnt_a, jnp.int32(0))
        wn_v = plsc.load_gather(ordv, [pv])
        dp_v = plsc.load_gather(ndrows, [av, jnp.full((_L,), _DP_LANE, jnp.int32)])
        sel = lanes == a
        dp_acc = jnp.where(sel, dp_v, dp_acc)
        idx_acc = jnp.where(sel, (base + a) * _B + wn_v, idx_acc)

    idxb[...] = idx_acc
    pltpu.async_copy(d_hbm.at[idxb], dnb, sem).wait()
    dn_acc = dnb[...]

    outv[...] = jnp.maximum(dp_acc - dn_acc + _MARGIN, 0.0)
    pltpu.sync_copy(outv.at[pl.ds(0, _NPW)], out_hbm.at[pl.ds(base, _NPW)])


@functools.lru_cache(maxsize=1)
def _make_mine():
    return pl.kernel(
        _mine_body,
        out_type=jax.ShapeDtypeStruct((_B,), jnp.float32),
        mesh=plsc.VectorSubcoreMesh(
            core_axis_name="c", subcore_axis_name="s",
            num_cores=_NC, num_subcores=_NS,
        ),
        compiler_params=pltpu.CompilerParams(needs_layout_passes=False),
        scratch_types=[
            pltpu.VMEM((_NPW, _L), jnp.float32),
            pltpu.VMEM((_NPW, _L), jnp.int32),
            pltpu.VMEM((_B,), jnp.int32),
            pltpu.VMEM((_L,), jnp.int32),
            pltpu.VMEM((_L,), jnp.float32),
            pltpu.VMEM((_L,), jnp.float32),
            pltpu.SemaphoreType.DMA,
        ],
    )


def _sum_kernel(h_ref, out_ref):
    h = h_ref[:, :]
    out_ref[:, :] = jnp.sum(h, axis=1, keepdims=True)


@jax.jit
def kernel(embeddings, labels):
    B = embeddings.shape[0]
    labels = labels.astype(jnp.int32)
    lab_row = labels.reshape(1, B)
    lab_col = labels.reshape(B, 1)
    d, nd, oc, ordc = pl.pallas_call(
        _dense_kernel,
        out_shape=[
            jax.ShapeDtypeStruct((B, B), jnp.float32),
            jax.ShapeDtypeStruct((B, _L), jnp.float32),
            jax.ShapeDtypeStruct((B, _L), jnp.int32),
            jax.ShapeDtypeStruct((B, 1), jnp.int32),
        ],
    )(embeddings, lab_row, lab_col)
    part = _make_mine()(d.reshape(B * B), nd, oc, ordc.reshape(B))
    out = pl.pallas_call(
        _sum_kernel,
        out_shape=jax.ShapeDtypeStruct((1, 1), jnp.float32),
    )(part.reshape(1, B))
    return out.reshape(())


# speedup vs baseline: 1.0553x; 1.0553x over previous
import functools

import jax
import jax.numpy as jnp
from jax import lax
from jax.experimental import pallas as pl
from jax.experimental.pallas import tpu as pltpu
from jax.experimental.pallas import tpu_sc as plsc

_MARGIN = 1.0
_NUM_CLASSES = 10
_B = 256
_BIG = 3.0e38
_NEG = -3.0e38

_NC = 2
_NS = 16
_L = 16
_NW = _NC * _NS
_NPW = _B // _NW
_DP_LANE = _NUM_CLASSES


def _dense_kernel(x_ref, lab_row_ref, lab_col_ref,
                  d_ref, nd_ref, oc_ref, ord_ref):
    x = x_ref[:, :]
    lab_row = lab_row_ref[:, :]
    lab_col = lab_col_ref[:, :]
    B = x.shape[0]

    g = jax.lax.dot_general(
        x, x, (((1,), (1,)), ((), ())), preferred_element_type=jnp.float32
    )
    eye = (
        jax.lax.broadcasted_iota(jnp.int32, (B, B), 0)
        == jax.lax.broadcasted_iota(jnp.int32, (B, B), 1)
    )
    diag = jnp.where(eye, g, 0.0)
    n_col = jnp.sum(diag, axis=1, keepdims=True)
    n_row = jnp.sum(diag, axis=0, keepdims=True)
    d = n_col + n_row - 2.0 * g
    d_ref[:, :] = d

    same = lab_col == lab_row
    lt = lab_col < lab_row
    gt = lab_col > lab_row

    dp = jnp.max(jnp.where(same, d, _NEG), axis=1, keepdims=True)

    s_row = jnp.sum(d, axis=1, keepdims=True)
    cols = [None] * _L
    for c in range(_NUM_CLASSES):
        cs_c = jnp.sum(jnp.where(lab_row == c, d, 0.0), axis=1, keepdims=True)
        cols[c] = s_row - cs_c
    cols[_DP_LANE] = dp
    for c in range(_DP_LANE + 1, _L):
        cols[c] = jnp.full((B, 1), _BIG, jnp.float32)
    nd_ref[:, :] = jnp.concatenate(cols, axis=1)

    off_col = jnp.sum(gt.astype(jnp.int32), axis=1, keepdims=True)
    cnt_col = jnp.sum(same.astype(jnp.int32), axis=1, keepdims=True)
    zeros = jnp.zeros((B, 1), jnp.int32)
    oc_ref[:, :] = jnp.concatenate([off_col, cnt_col] + [zeros] * (_L - 2),
                                   axis=1)

    idx_row = jax.lax.broadcasted_iota(jnp.int32, (B, B), 1)
    idx_col = jax.lax.broadcasted_iota(jnp.int32, (B, B), 0)
    before = jnp.logical_or(lt, jnp.logical_and(same, idx_col < idx_row))
    pos_row = jnp.sum(before.astype(jnp.int32), axis=0, keepdims=True)

    sel = pos_row == idx_col
    ord_ref[:, :] = jnp.sum(jnp.where(sel, idx_row, 0), axis=1, keepdims=True)


def _mine_body(d_hbm, nd_hbm, oc_hbm, ord_hbm, out_hbm,
               drows, ndrows, ocrows, ordv, outv, sem):
    c = lax.axis_index("c")
    s = lax.axis_index("s")
    wid = s * _NC + c
    base = wid * _NPW

    cp1 = pltpu.make_async_copy(d_hbm.at[pl.ds(base, _NPW)], drows, sem)
    cp2 = pltpu.make_async_copy(nd_hbm.at[pl.ds(base, _NPW)], ndrows, sem)
    cp3 = pltpu.make_async_copy(oc_hbm.at[pl.ds(base, _NPW)], ocrows, sem)
    cp4 = pltpu.make_async_copy(ord_hbm, ordv, sem)
    cp1.start()
    cp2.start()
    cp3.start()
    cp4.start()
    cp1.wait()
    cp2.wait()
    cp3.wait()
    cp4.wait()

    lanes = jax.lax.broadcasted_iota(jnp.int32, (_L,), 0)
    hb = jnp.zeros((_L,), jnp.float32)

    for a in range(_NPW):
        av = jnp.full((_L,), a, dtype=jnp.int32)
        ndv = ndrows[a, pl.ds(0, _L)]
        neg_v = jnp.where(lanes < _NUM_CLASSES, ndv, _BIG)
        minv = jnp.min(neg_v)
        m_v = plsc.all_reduce_ffs(neg_v == minv)
        off_a = plsc.load_gather(ocrows, [av, jnp.zeros((_L,), jnp.int32)])
        cnt_a = plsc.load_gather(ocrows, [av, jnp.ones((_L,), jnp.int32)])
        pv = m_v + jnp.where(m_v >= off_a, cnt_a, jnp.int32(0))
        wn_v = plsc.load_gather(ordv, [pv])
        dn_v = plsc.load_gather(drows, [av, wn_v])
        dp_v = plsc.load_gather(ndrows, [av, jnp.full((_L,), _DP_LANE, jnp.int32)])
        hinge = jnp.maximum(dp_v - dn_v + _MARGIN, 0.0)
        hb = jnp.where(lanes == a, hinge, hb)

    outv[...] = hb
    pltpu.sync_copy(outv.at[pl.ds(0, _NPW)], out_hbm.at[pl.ds(base, _NPW)])


@functools.lru_cache(maxsize=1)
def _make_mine():
    return pl.kernel(
        _mine_body,
        out_type=jax.ShapeDtypeStruct((_B,), jnp.float32),
        mesh=plsc.VectorSubcoreMesh(
            core_axis_name="c", subcore_axis_name="s",
            num_cores=_NC, num_subcores=_NS,
        ),
        compiler_params=pltpu.CompilerParams(needs_layout_passes=False),
        scratch_types=[
            pltpu.VMEM((_NPW, _B), jnp.float32),
            pltpu.VMEM((_NPW, _L), jnp.float32),
            pltpu.VMEM((_NPW, _L), jnp.int32),
            pltpu.VMEM((_B,), jnp.int32),
            pltpu.VMEM((_L,), jnp.float32),
            pltpu.SemaphoreType.DMA,
        ],
    )


def _sum_kernel(h_ref, out_ref):
    h = h_ref[:, :]
    out_ref[:, :] = jnp.sum(h, axis=1, keepdims=True)


@jax.jit
def kernel(embeddings, labels):
    B = embeddings.shape[0]
    labels = labels.astype(jnp.int32)
    lab_row = labels.reshape(1, B)
    lab_col = labels.reshape(B, 1)
    d, nd, oc, ordc = pl.pallas_call(
        _dense_kernel,
        out_shape=[
            jax.ShapeDtypeStruct((B, B), jnp.float32),
            jax.ShapeDtypeStruct((B, _L), jnp.float32),
            jax.ShapeDtypeStruct((B, _L), jnp.int32),
            jax.ShapeDtypeStruct((B, 1), jnp.int32),
        ],
    )(embeddings, lab_row, lab_col)
    part = _make_mine()(d, nd, oc, ordc.reshape(B))
    out = pl.pallas_call(
        _sum_kernel,
        out_shape=jax.ShapeDtypeStruct((1, 1), jnp.float32),
    )(part.reshape(1, B))
    return out.reshape(())


# speedup vs baseline: 1.1378x; 1.0782x over previous
import functools

import jax
import jax.numpy as jnp
from jax import lax
from jax.experimental import pallas as pl
from jax.experimental.pallas import tpu as pltpu
from jax.experimental.pallas import tpu_sc as plsc

_MARGIN = 1.0
_NUM_CLASSES = 10
_B = 256
_BIG = 3.0e38
_NEG = -3.0e38

_NC = 1
_NS = 16
_L = 16
_NW = _NC * _NS
_NPW = _B // _NW
_DP_LANE = _NUM_CLASSES


def _dense_kernel(x_ref, lab_row_ref, lab_col_ref,
                  d_ref, nd_ref, oc_ref, ord_ref):
    x = x_ref[:, :]
    lab_row = lab_row_ref[:, :]
    lab_col = lab_col_ref[:, :]
    B = x.shape[0]

    g = jax.lax.dot_general(
        x, x, (((1,), (1,)), ((), ())), preferred_element_type=jnp.float32
    )
    eye = (
        jax.lax.broadcasted_iota(jnp.int32, (B, B), 0)
        == jax.lax.broadcasted_iota(jnp.int32, (B, B), 1)
    )
    diag = jnp.where(eye, g, 0.0)
    n_col = jnp.sum(diag, axis=1, keepdims=True)
    n_row = jnp.sum(diag, axis=0, keepdims=True)
    d = n_col + n_row - 2.0 * g
    d_ref[:, :] = d

    same = lab_col == lab_row
    lt = lab_col < lab_row
    gt = lab_col > lab_row

    dp = jnp.max(jnp.where(same, d, _NEG), axis=1, keepdims=True)

    s_row = jnp.sum(d, axis=1, keepdims=True)
    cols = [None] * _L
    for c in range(_NUM_CLASSES):
        cs_c = jnp.sum(jnp.where(lab_row == c, d, 0.0), axis=1, keepdims=True)
        cols[c] = s_row - cs_c
    cols[_DP_LANE] = dp
    for c in range(_DP_LANE + 1, _L):
        cols[c] = jnp.full((B, 1), _BIG, jnp.float32)
    nd_ref[:, :] = jnp.concatenate(cols, axis=1)

    off_col = jnp.sum(gt.astype(jnp.int32), axis=1, keepdims=True)
    cnt_col = jnp.sum(same.astype(jnp.int32), axis=1, keepdims=True)
    zeros = jnp.zeros((B, 1), jnp.int32)
    oc_ref[:, :] = jnp.concatenate([off_col, cnt_col] + [zeros] * (_L - 2),
                                   axis=1)

    idx_row = jax.lax.broadcasted_iota(jnp.int32, (B, B), 1)
    idx_col = jax.lax.broadcasted_iota(jnp.int32, (B, B), 0)
    before = jnp.logical_or(lt, jnp.logical_and(same, idx_col < idx_row))
    pos_row = jnp.sum(before.astype(jnp.int32), axis=0, keepdims=True)

    sel = pos_row == idx_col
    ord_ref[:, :] = jnp.sum(jnp.where(sel, idx_row, 0), axis=1, keepdims=True)


def _mine_body(d_hbm, nd_hbm, oc_hbm, ord_hbm, out_hbm,
               drows, ndrows, ocrows, ordv, outv, sem):
    c = lax.axis_index("c")
    s = lax.axis_index("s")
    wid = s * _NC + c
    base = wid * _NPW

    cp1 = pltpu.make_async_copy(d_hbm.at[pl.ds(base, _NPW)], drows, sem)
    cp2 = pltpu.make_async_copy(nd_hbm.at[pl.ds(base, _NPW)], ndrows, sem)
    cp3 = pltpu.make_async_copy(oc_hbm.at[pl.ds(base, _NPW)], ocrows, sem)
    cp4 = pltpu.make_async_copy(ord_hbm, ordv, sem)
    cp1.start()
    cp2.start()
    cp3.start()
    cp4.start()
    cp1.wait()
    cp2.wait()
    cp3.wait()
    cp4.wait()

    lanes = jax.lax.broadcasted_iota(jnp.int32, (_L,), 0)
    hb = jnp.zeros((_L,), jnp.float32)

    for a in range(_NPW):
        av = jnp.full((_L,), a, dtype=jnp.int32)
        ndv = ndrows[a, pl.ds(0, _L)]
        neg_v = jnp.where(lanes < _NUM_CLASSES, ndv, _BIG)
        minv = jnp.min(neg_v)
        m_v = plsc.all_reduce_ffs(neg_v == minv)
        off_a = plsc.load_gather(ocrows, [av, jnp.zeros((_L,), jnp.int32)])
        cnt_a = plsc.load_gather(ocrows, [av, jnp.ones((_L,), jnp.int32)])
        pv = m_v + jnp.where(m_v >= off_a, cnt_a, jnp.int32(0))
        wn_v = plsc.load_gather(ordv, [pv])
        dn_v = plsc.load_gather(drows, [av, wn_v])
        dp_v = plsc.load_gather(ndrows, [av, jnp.full((_L,), _DP_LANE, jnp.int32)])
        hinge = jnp.maximum(dp_v - dn_v + _MARGIN, 0.0)
        hb = jnp.where(lanes == a, hinge, hb)

    outv[...] = hb
    pltpu.sync_copy(outv.at[pl.ds(0, _NPW)], out_hbm.at[pl.ds(base, _NPW)])


@functools.lru_cache(maxsize=1)
def _make_mine():
    return pl.kernel(
        _mine_body,
        out_type=jax.ShapeDtypeStruct((_B,), jnp.float32),
        mesh=plsc.VectorSubcoreMesh(
            core_axis_name="c", subcore_axis_name="s",
            num_cores=_NC, num_subcores=_NS,
        ),
        compiler_params=pltpu.CompilerParams(needs_layout_passes=False),
        scratch_types=[
            pltpu.VMEM((_NPW, _B), jnp.float32),
            pltpu.VMEM((_NPW, _L), jnp.float32),
            pltpu.VMEM((_NPW, _L), jnp.int32),
            pltpu.VMEM((_B,), jnp.int32),
            pltpu.VMEM((_L,), jnp.float32),
            pltpu.SemaphoreType.DMA,
        ],
    )


def _sum_kernel(h_ref, out_ref):
    h = h_ref[:, :]
    out_ref[:, :] = jnp.sum(h, axis=1, keepdims=True)


@jax.jit
def kernel(embeddings, labels):
    B = embeddings.shape[0]
    labels = labels.astype(jnp.int32)
    lab_row = labels.reshape(1, B)
    lab_col = labels.reshape(B, 1)
    d, nd, oc, ordc = pl.pallas_call(
        _dense_kernel,
        out_shape=[
            jax.ShapeDtypeStruct((B, B), jnp.float32),
            jax.ShapeDtypeStruct((B, _L), jnp.float32),
            jax.ShapeDtypeStruct((B, _L), jnp.int32),
            jax.ShapeDtypeStruct((B, 1), jnp.int32),
        ],
    )(embeddings, lab_row, lab_col)
    part = _make_mine()(d, nd, oc, ordc.reshape(B))
    out = pl.pallas_call(
        _sum_kernel,
        out_shape=jax.ShapeDtypeStruct((1, 1), jnp.float32),
    )(part.reshape(1, B))
    return out.reshape(())


# speedup vs baseline: 1.1509x; 1.0115x over previous
import functools

import jax
import jax.numpy as jnp
from jax import lax
from jax.experimental import pallas as pl
from jax.experimental.pallas import tpu as pltpu
from jax.experimental.pallas import tpu_sc as plsc

_MARGIN = 1.0
_NUM_CLASSES = 10
_B = 256
_BIG = 3.0e38
_NEG = -3.0e38

_NC = 1
_NS = 16
_L = 16
_NW = _NC * _NS
_NPW = _B // _NW
_DP_LANE = _NUM_CLASSES


def _dense_kernel(x_ref, lab_row_ref, lab_col_ref,
                  d_ref, nd_ref, oc_ref, ord_ref):
    x = x_ref[:, :]
    lab_row = lab_row_ref[:, :]
    lab_col = lab_col_ref[:, :]
    B = x.shape[0]

    g = jax.lax.dot_general(
        x, x, (((1,), (1,)), ((), ())), preferred_element_type=jnp.float32
    )
    eye = (
        jax.lax.broadcasted_iota(jnp.int32, (B, B), 0)
        == jax.lax.broadcasted_iota(jnp.int32, (B, B), 1)
    )
    diag = jnp.where(eye, g, 0.0)
    n_col = jnp.sum(diag, axis=1, keepdims=True)
    n_row = jnp.sum(diag, axis=0, keepdims=True)
    d = n_col + n_row - 2.0 * g
    d_ref[:, :] = d

    same = lab_col == lab_row
    lt = lab_col < lab_row
    gt = lab_col > lab_row

    dp = jnp.max(jnp.where(same, d, _NEG), axis=1, keepdims=True)

    s_row = jnp.sum(d, axis=1, keepdims=True)
    cols = [None] * _L
    for c in range(_NUM_CLASSES):
        cs_c = jnp.sum(jnp.where(lab_row == c, d, 0.0), axis=1, keepdims=True)
        cols[c] = s_row - cs_c
    cols[_DP_LANE] = dp
    for c in range(_DP_LANE + 1, _L):
        cols[c] = jnp.full((B, 1), _BIG, jnp.float32)
    nd_ref[:, :] = jnp.concatenate(cols, axis=1)

    off_col = jnp.sum(gt.astype(jnp.int32), axis=1, keepdims=True)
    cnt_col = jnp.sum(same.astype(jnp.int32), axis=1, keepdims=True)
    zeros = jnp.zeros((B, 1), jnp.int32)
    oc_ref[:, :] = jnp.concatenate([off_col, cnt_col] + [zeros] * (_L - 2),
                                   axis=1)

    idx_row = jax.lax.broadcasted_iota(jnp.int32, (B, B), 1)
    idx_col = jax.lax.broadcasted_iota(jnp.int32, (B, B), 0)
    before = jnp.logical_or(lt, jnp.logical_and(same, idx_col < idx_row))
    pos_row = jnp.sum(before.astype(jnp.int32), axis=0, keepdims=True)

    sel = pos_row == idx_col
    ord_ref[:, :] = jnp.sum(jnp.where(sel, idx_row, 0), axis=1, keepdims=True)


def _mine_body(d_hbm, nd_hbm, oc_hbm, ord_hbm, out_hbm,
               drows, ndrows, ocrows, ordv, outv, semd, sem):
    c = lax.axis_index("c")
    s = lax.axis_index("s")
    wid = s * _NC + c
    base = wid * _NPW

    cpd = pltpu.make_async_copy(d_hbm.at[pl.ds(base, _NPW)], drows, semd)
    cp2 = pltpu.make_async_copy(nd_hbm.at[pl.ds(base, _NPW)], ndrows, sem)
    cp3 = pltpu.make_async_copy(oc_hbm.at[pl.ds(base, _NPW)], ocrows, sem)
    cp4 = pltpu.make_async_copy(ord_hbm, ordv, sem)
    cpd.start()
    cp2.start()
    cp3.start()
    cp4.start()
    cp2.wait()
    cp3.wait()
    cp4.wait()

    lanes = jax.lax.broadcasted_iota(jnp.int32, (_L,), 0)
    m_acc = jnp.zeros((_L,), jnp.int32)

    for a in range(_NPW):
        ndv = ndrows[a, pl.ds(0, _L)]
        neg_v = jnp.where(lanes < _NUM_CLASSES, ndv, _BIG)
        minv = jnp.min(neg_v)
        m_v = plsc.all_reduce_ffs(neg_v == minv)
        m_acc = jnp.where(lanes == a, m_v, m_acc)

    zeros = jnp.zeros((_L,), jnp.int32)
    off_all = plsc.load_gather(ocrows, [lanes, zeros])
    cnt_all = plsc.load_gather(ocrows, [lanes, jnp.ones((_L,), jnp.int32)])
    dp_all = plsc.load_gather(ndrows, [lanes, jnp.full((_L,), _DP_LANE, jnp.int32)])
    pv = m_acc + jnp.where(m_acc >= off_all, cnt_all, zeros)
    wn_all = plsc.load_gather(ordv, [pv])
    cpd.wait()
    dn_all = plsc.load_gather(drows, [lanes, wn_all])

    outv[...] = jnp.maximum(dp_all - dn_all + _MARGIN, 0.0)
    pltpu.sync_copy(outv.at[pl.ds(0, _NPW)], out_hbm.at[pl.ds(base, _NPW)])


@functools.lru_cache(maxsize=1)
def _make_mine():
    return pl.kernel(
        _mine_body,
        out_type=jax.ShapeDtypeStruct((_B,), jnp.float32),
        mesh=plsc.VectorSubcoreMesh(
            core_axis_name="c", subcore_axis_name="s",
            num_cores=_NC, num_subcores=_NS,
        ),
        compiler_params=pltpu.CompilerParams(needs_layout_passes=False),
        scratch_types=[
            pltpu.VMEM((_NPW, _B), jnp.float32),
            pltpu.VMEM((_NPW, _L), jnp.float32),
            pltpu.VMEM((_NPW, _L), jnp.int32),
            pltpu.VMEM((_B,), jnp.int32),
            pltpu.VMEM((_L,), jnp.float32),
            pltpu.SemaphoreType.DMA,
            pltpu.SemaphoreType.DMA,
        ],
    )


def _sum_kernel(h_ref, out_ref):
    h = h_ref[:, :]
    out_ref[:, :] = jnp.sum(h, axis=1, keepdims=True)


@jax.jit
def kernel(embeddings, labels):
    B = embeddings.shape[0]
    labels = labels.astype(jnp.int32)
    lab_row = labels.reshape(1, B)
    lab_col = labels.reshape(B, 1)
    d, nd, oc, ordc = pl.pallas_call(
        _dense_kernel,
        out_shape=[
            jax.ShapeDtypeStruct((B, B), jnp.float32),
            jax.ShapeDtypeStruct((B, _L), jnp.float32),
            jax.ShapeDtypeStruct((B, _L), jnp.int32),
            jax.ShapeDtypeStruct((B, 1), jnp.int32),
        ],
    )(embeddings, lab_row, lab_col)
    part = _make_mine()(d, nd, oc, ordc.reshape(B))
    out = pl.pallas_call(
        _sum_kernel,
        out_shape=jax.ShapeDtypeStruct((1, 1), jnp.float32),
    )(part.reshape(1, B))
    return out.reshape(())
